# baseline (speedup 1.0000x reference)
import jax
import jax.numpy as jnp
from jax import lax
from jax.experimental import pallas as pl
from jax.experimental.pallas import tpu as pltpu, tpu_sc as plsc

V = 1000000
D = 64
B = 16384
L = 50

NC = 2
NS = 16
LANES = 16
NW = NC * NS
BC = B // NW
NCH = BC // 128
NWAVE = 2
VT = V // 128
TPW = VT // NW + 1

_CP = dict(needs_layout_passes=False, use_tc_tiling_on_sc=True)


def _rsqrt(x):
    i = lax.bitcast_convert_type(x, jnp.int32)
    y = lax.bitcast_convert_type(
        jnp.int32(0x5F3759DF) - lax.shift_right_arithmetic(i, 1), jnp.float32)
    for _ in range(3):
        y = y * (1.5 - 0.5 * x * y * y)
    return y


def _transpose_block(src_v, dst_v, lanes, nv):
    def vstep(v0, _):
        vvec = v0 * LANES + lanes
        lv = lax.shift_right_logical(vvec, 1)
        pof = lax.shift_left(vvec & 1, 6)
        for d0 in range(D):
            dvec = (d0 + lanes) & (D - 1)
            val = plsc.load_gather(src_v, [dvec, vvec])
            plsc.store_scatter(dst_v, [lv, pof + dvec], val)
        return ()

    lax.fori_loop(0, nv, vstep, (), unroll=False)


def _k1_body(win_t, wout_t, tin2, tout2, comb,
             r0, r1, r2, r3, l0, l1,
             sr0, sr1, sr2, sr3, sw0, sw1):
    wid = lax.axis_index("s") * NC + lax.axis_index("c")
    lanes = lax.iota(jnp.int32, LANES)
    RB, SR = [r0, r1, r2, r3], [sr0, sr1, sr2, sr3]
    LB, SW = [l0, l1], [sw0, sw1]
    PR = 4
    nt = (VT - wid + NW - 1) // NW

    for tab, w_t, tl2 in ((0, win_t, tin2), (1, wout_t, tout2)):
        obase = tab * (V // 2)

        def read(t, p):
            pltpu.async_copy(w_t.at[:, pl.ds(t * 128, 128)], RB[p], SR[p])

        def wait_read(t, p):
            pltpu.make_async_copy(w_t.at[:, pl.ds(t * 128, 128)], RB[p],
                                  SR[p]).wait()

        def write(t, q):
            pltpu.async_copy(LB[q], comb.at[pl.ds(obase + t * 64, 64), :],
                             SW[q])

        def drain_write(q):
            pltpu.make_async_copy(LB[q], comb.at[pl.ds(obase, 64), :],
                                  SW[q]).wait()

        for p in range(PR):
            read(wid + p * NW, p)

        def step(s, _):
            for p in range(PR):
                i = s * PR + p
                t = wid + i * NW

                @pl.when(i < nt)
                def _():
                    wait_read(t, p)

                    @pl.when(i >= 2)
                    def _():
                        drain_write(p % 2)
                    _transpose_block(RB[p], LB[p % 2], lanes, 8)

                    @pl.when(i + PR < nt)
                    def _():
                        read(t + PR * NW, p)
                    write(t, p % 2)
            return ()

        lax.fori_loop(0, (TPW + PR - 1) // PR, step, (), unroll=False)
        drain_write(0)
        drain_write(1)

        @pl.when(wid == 0)
        def _():
            pltpu.sync_copy(tl2, l0.at[pl.ds(0, 32), :])
            pltpu.sync_copy(l0.at[pl.ds(0, 32), :],
                            comb.at[pl.ds(obase + VT * 64, 32), :])


def _k2_body(cen_hbm, ctx_hbm, comb, out_hbm,
             ridx_v, hidx_v, poff_v, in_v, wave_v, invin_v, res_v, sem):
    wid = lax.axis_index("s") * NC + lax.axis_index("c")
    base = wid * BC
    lanes = lax.iota(jnp.int32, LANES)

    def halve_indices(off):
        for j in range(NCH):
            for k in range(8):
                v = ridx_v[j, pl.ds(k * LANES, LANES)]
                hidx_v[j, pl.ds(k * LANES, LANES)] = (
                    lax.shift_right_logical(v, 1) + off)
                poff_v[pl.ds((j * 8 + k) * LANES, LANES)] = lax.shift_left(
                    v & 1, 6)

    for j in range(NCH):
        pltpu.sync_copy(cen_hbm.at[pl.ds(base + j * 128, 128)], ridx_v.at[j])
    halve_indices(0)
    for w in range(NCH // 2):
        for j in range(2):
            pltpu.async_copy(comb.at[hidx_v.at[w * 2 + j]],
                             wave_v.at[pl.ds(j * 128, 128), :], sem)
        for j in range(2):
            pltpu.make_async_copy(comb.at[hidx_v.at[w * 2 + j]],
                                  wave_v.at[pl.ds(j * 128, 128), :], sem).wait()

        def cgrp(g, _):
            rows = g * LANES + lanes
            gpos = w * 256 + g * LANES + lanes
            po = plsc.load_gather(poff_v, [gpos])
            acc = jnp.zeros((LANES,), jnp.float32)
            for d in range(D):
                col = (lanes + d) & (D - 1)
                v = plsc.load_gather(wave_v, [rows, col + po])
                plsc.store_scatter(in_v, [gpos, col], v)
                acc += v * v
            invin_v[pl.ds(w * 256 + g * LANES, LANES)] = _rsqrt(acc)
            return ()

        lax.fori_loop(0, 256 // LANES, cgrp, (), unroll=False)

    def l_body(l, _):
        for j in range(NCH):
            pltpu.sync_copy(ctx_hbm.at[l, pl.ds(base + j * 128, 128)],
                            ridx_v.at[j])
        halve_indices(V // 2)

        for w in range(NWAVE):
            for j in range(2):
                pltpu.async_copy(comb.at[hidx_v.at[w * 2 + j]],
                                 wave_v.at[pl.ds(j * 128, 128), :], sem)
            for j in range(2):
                pltpu.make_async_copy(comb.at[hidx_v.at[w * 2 + j]],
                                      wave_v.at[pl.ds(j * 128, 128), :],
                                      sem).wait()

            def g_body(g, _):
                rows = g * LANES + lanes
                gpos = w * 256 + g * LANES + lanes
                po = plsc.load_gather(poff_v, [gpos])
                acc_d = jnp.zeros((LANES,), jnp.float32)
                acc_s = jnp.zeros((LANES,), jnp.float32)
                for d in range(D):
                    col = (lanes + d) & (D - 1)
                    o = plsc.load_gather(wave_v, [rows, col + po])
                    i = plsc.load_gather(in_v, [gpos, col])
                    acc_d += o * i
                    acc_s += o * o
                res = (acc_d * _rsqrt(acc_s)
                       * invin_v[pl.ds(w * 256 + g * LANES, LANES)])
                res_v[pl.ds(w * 256 + g * LANES, LANES)] = res
                return ()

            lax.fori_loop(0, 256 // LANES, g_body, (), unroll=False)

        pltpu.sync_copy(res_v, out_hbm.at[l, pl.ds(base, BC)])
        return ()

    lax.fori_loop(0, L, l_body, (), unroll=False)


@jax.jit
def kernel(center, context, emb_in_weight, emb_out_weight):
    mesh = plsc.VectorSubcoreMesh(core_axis_name="c", subcore_axis_name="s")

    k1 = pl.kernel(
        _k1_body,
        out_type=jax.ShapeDtypeStruct((V, 2 * D), jnp.float32),
        mesh=mesh,
        compiler_params=pltpu.CompilerParams(**_CP),
        scratch_types=(
            [pltpu.VMEM((D, 128), jnp.float32)] * 6
            + [pltpu.SemaphoreType.DMA] * 6),
    )
    tin2 = emb_in_weight[VT * 128:, :].reshape(32, 2 * D)
    tout2 = emb_out_weight[VT * 128:, :].reshape(32, 2 * D)
    comb = k1(emb_in_weight.T, emb_out_weight.T, tin2, tout2)

    k2 = pl.kernel(
        _k2_body,
        out_type=jax.ShapeDtypeStruct((L, B), jnp.float32),
        mesh=mesh,
        compiler_params=pltpu.CompilerParams(**_CP),
        scratch_types=[
            pltpu.VMEM((NCH, 128), jnp.int32),
            pltpu.VMEM((NCH, 128), jnp.int32),
            pltpu.VMEM((BC,), jnp.int32),
            pltpu.VMEM((BC, D), jnp.float32),
            pltpu.VMEM((256, 2 * D), jnp.float32),
            pltpu.VMEM((BC,), jnp.float32),
            pltpu.VMEM((BC,), jnp.float32),
            pltpu.SemaphoreType.DMA,
        ],
    )
    return k2(center, context, comb)


# speedup vs baseline: 1.3749x; 1.3749x over previous
import jax
import jax.numpy as jnp
from jax import lax
from jax.experimental import pallas as pl
from jax.experimental.pallas import tpu as pltpu, tpu_sc as plsc

SIZE_VOCAB = 1000000
D = 64
B = 16384
L = 50

NC = 2
NS = 16
LANES = 16
NW = NC * NS
BC = B // NW
NCH = BC // 128


def _rsqrt(x):
    i = lax.bitcast_convert_type(x, jnp.int32)
    y = lax.bitcast_convert_type(
        jnp.int32(0x5F3759DF) - lax.shift_right_arithmetic(i, 1), jnp.float32)
    for _ in range(3):
        y = y * (1.5 - 0.5 * x * y * y)
    return y


def _body(cen_hbm, ctx_hbm, win_hbm, wout_hbm, out_hbm,
          cidx_v, ctxidx_v, in_v, out_v, invin_v, res_v, sem):
    wid = lax.axis_index("s") * NC + lax.axis_index("c")
    base = wid * BC

    lanes = lax.iota(jnp.int32, LANES)

    pltpu.sync_copy(cen_hbm.at[wid], cidx_v)
    pltpu.sync_copy(ctx_hbm.at[wid], ctxidx_v)

    for j in range(NCH):
        pltpu.async_copy(win_hbm.at[cidx_v.at[j]],
                         in_v.at[pl.ds(j * 128, 128), :], sem)
    for j in range(NCH):
        pltpu.make_async_copy(win_hbm.at[cidx_v.at[j]],
                              in_v.at[pl.ds(j * 128, 128), :], sem).wait()

    def norm_body(g, _):
        rows = g * LANES + lanes
        acc = jnp.zeros((LANES,), jnp.float32)
        for d in range(D):
            col = (lanes + d) & (D - 1)
            v = plsc.load_gather(in_v, [rows, col])
            acc += v * v
        invin_v[pl.ds(g * LANES, LANES)] = _rsqrt(acc)
        return ()

    lax.fori_loop(0, BC // LANES, norm_body, (), unroll=False)

    def l_body(l, _):
        for j in range(NCH):
            pltpu.async_copy(wout_hbm.at[ctxidx_v.at[l, j]],
                             out_v.at[pl.ds(j * 128, 128), :], sem)
        for j in range(NCH):
            pltpu.make_async_copy(wout_hbm.at[ctxidx_v.at[l, j]],
                                  out_v.at[pl.ds(j * 128, 128), :],
                                  sem).wait()

        def g_body(g, _):
            rows = g * LANES + lanes
            acc_d = jnp.zeros((LANES,), jnp.float32)
            acc_s = jnp.zeros((LANES,), jnp.float32)
            for d in range(D):
                col = (lanes + d) & (D - 1)
                o = plsc.load_gather(out_v, [rows, col])
                i = plsc.load_gather(in_v, [rows, col])
                acc_d += o * i
                acc_s += o * o
            res = acc_d * _rsqrt(acc_s) * invin_v[pl.ds(g * LANES, LANES)]
            res_v[pl.ds(g * LANES, LANES)] = res
            return ()

        lax.fori_loop(0, BC // LANES, g_body, (), unroll=False)
        pltpu.sync_copy(res_v, out_hbm.at[l, pl.ds(base, BC)])
        return ()

    lax.fori_loop(0, L, l_body, (), unroll=False)


@jax.jit
def kernel(center, context, emb_in_weight, emb_out_weight):
    cen = center.reshape(NW, NCH, 128).astype(jnp.int32)
    ctx = (context.reshape(L, NW, BC).transpose(1, 0, 2)
           .reshape(NW, L, NCH, 128).astype(jnp.int32))

    mesh = plsc.VectorSubcoreMesh(core_axis_name="c", subcore_axis_name="s")
    f = pl.kernel(
        _body,
        out_type=jax.ShapeDtypeStruct((L, B), jnp.float32),
        mesh=mesh,
        compiler_params=pltpu.CompilerParams(
            needs_layout_passes=False, use_tc_tiling_on_sc=False),
        scratch_types=[
            pltpu.VMEM((NCH, 128), jnp.int32),
            pltpu.VMEM((L, NCH, 128), jnp.int32),
            pltpu.VMEM((BC, D), jnp.float32),
            pltpu.VMEM((BC, D), jnp.float32),
            pltpu.VMEM((BC,), jnp.float32),
            pltpu.VMEM((BC,), jnp.float32),
            pltpu.SemaphoreType.DMA,
        ],
    )
    return f(cen, ctx, emb_in_weight, emb_out_weight)
